# Initial kernel scaffold; baseline (speedup 1.0000x reference)
import jax
import jax.numpy as jnp
from jax.experimental import pallas as pl


def kernel(x, edge_index, batch, W1, b1, W2, b2, W3, b3, l1W, l1b, l2W, l2b):
    raise NotImplementedError("write your pallas kernel here")


# speedup vs baseline: 9.9248x; 9.9248x over previous
import functools

import jax
import jax.numpy as jnp
from jax import lax
from jax.experimental import pallas as pl
from jax.experimental.pallas import tpu as pltpu
from jax.experimental.pallas import tpu_sc as plsc

NN = 50000
EE = 800000
HH = 64
GG = 512

HALF = 25088
NP_ = 2 * HALF
NR = NP_ // 128
TPR = 1568
EP = 802816
ER = EP // 128
TER = ER // 16
TRASH = HALF
BIGDST = 0x3FFFFF0

_mesh = plsc.VectorSubcoreMesh(core_axis_name="c", subcore_axis_name="s")


def _zero_fill_rows(ref, nrows):
    def body(i, _):
        ref[i // 4, pl.ds((i % 4) * 16, 16)] = jnp.zeros((16,), jnp.float32)
        return ()
    lax.fori_loop(0, nrows * 4, body, ())


@functools.partial(
    pl.kernel,
    out_type=(
        jax.ShapeDtypeStruct((2, ER, 128), jnp.int32),
        jax.ShapeDtypeStruct((NP_, 16), jnp.float32),
        jax.ShapeDtypeStruct((2, GG, 16), jnp.float32),
    ),
    mesh=_mesh,
    compiler_params=pltpu.CompilerParams(use_tc_tiling_on_sc=False),
    scratch_types=[
        pltpu.VMEM((8, 128), jnp.int32),
        pltpu.VMEM((8, 128), jnp.int32),
        pltpu.VMEM((32, 128), jnp.int32),
        pltpu.VMEM((128, 16), jnp.float32),
        pltpu.VMEM((176, 16), jnp.float32),
        pltpu.VMEM_SHARED((HALF + 16, 16), jnp.float32),
        pltpu.VMEM_SHARED((GG + 16, 16), jnp.float32),
    ],
)
def _sc_prep(dst_hbm, batch_hbm, dstrel_hbm, deg_hbm, cnt_hbm,
             dvm, rvm, bvm, ones, zeros, deg_acc, cnt_acc):
    c = lax.axis_index("c")
    s = lax.axis_index("s")

    def f_ones(i, _):
        ones[i, :] = jnp.full((16,), 1.0, jnp.float32)
        return ()
    lax.fori_loop(0, 128, f_ones, ())

    def f_zero(i, _):
        zeros[i, :] = jnp.zeros((16,), jnp.float32)
        return ()
    lax.fori_loop(0, 176, f_zero, ())

    off = 0
    while off < TPR:
        sz = min(176, TPR - off)
        pltpu.sync_copy(zeros.at[pl.ds(0, sz)],
                        deg_acc.at[pl.ds(s * TPR + off, sz)])
        off += sz

    pltpu.sync_copy(zeros.at[pl.ds(0, 32)], cnt_acc.at[pl.ds(s * 32, 32)])

    @pl.when(s == 15)
    def _():
        pltpu.sync_copy(zeros.at[pl.ds(0, 16)], deg_acc.at[pl.ds(HALF, 16)])
        pltpu.sync_copy(zeros.at[pl.ds(0, 16)], cnt_acc.at[pl.ds(GG, 16)])
    plsc.subcore_barrier()

    def chunk(i, _):
        r0 = s * TER + i * 8
        pltpu.sync_copy(dst_hbm.at[pl.ds(r0, 8)], dvm)
        for j in range(8):
            for k in range(8):
                d = dvm[j, pl.ds(k * 16, 16)]
                rel = d - c * HALF
                ok = (rel >= 0) & (rel < HALF)
                rvm[j, pl.ds(k * 16, 16)] = jnp.where(ok, rel, TRASH)
        pltpu.sync_copy(rvm, dstrel_hbm.at[c, pl.ds(r0, 8)])
        for j in range(8):
            pltpu.sync_copy(ones, deg_acc.at[rvm.at[j]], add=True)
        return ()
    lax.fori_loop(0, TER // 8, chunk, ())

    pltpu.sync_copy(batch_hbm.at[pl.ds(s * 24, 24)], bvm.at[pl.ds(0, 24)])
    for j in range(24):
        pltpu.sync_copy(ones, cnt_acc.at[bvm.at[j]], add=True)

    @pl.when(s == 15)
    def _():
        pltpu.sync_copy(batch_hbm.at[pl.ds(384, 8)], bvm.at[pl.ds(24, 8)])
        for j in range(24, 32):
            pltpu.sync_copy(ones, cnt_acc.at[bvm.at[j]], add=True)
    plsc.subcore_barrier()

    pltpu.sync_copy(deg_acc.at[pl.ds(s * TPR, TPR)],
                    deg_hbm.at[pl.ds(c * HALF + s * TPR, TPR)])

    @pl.when(s == 0)
    def _():
        pltpu.sync_copy(cnt_acc.at[pl.ds(0, GG)], cnt_hbm.at[c])


@functools.partial(
    pl.kernel,
    out_type=jax.ShapeDtypeStruct((NP_, 64), jnp.float32),
    mesh=_mesh,
    compiler_params=pltpu.CompilerParams(use_tc_tiling_on_sc=False),
    scratch_types=[
        pltpu.VMEM((2, 128), jnp.int32),
        pltpu.VMEM((2, 128), jnp.int32),
        pltpu.VMEM((256, 64), jnp.float32),
        pltpu.VMEM_SHARED((HALF + 16, 64), jnp.float32),
        pltpu.SemaphoreType.DMA,
    ],
)
def _sc_prop(y_hbm, src_hbm, dstrel_hbm, z_hbm, svm, dvm, rows, acc, sem):
    c = lax.axis_index("c")
    s = lax.axis_index("s")

    _zero_fill_rows(rows, 256)
    for off, sz in ((0, 256), (256, 256), (512, 256), (768, 256),
                    (1024, 256), (1280, 256), (1536, 32)):
        pltpu.sync_copy(rows.at[pl.ds(0, sz)], acc.at[pl.ds(s * TPR + off, sz)])

    @pl.when(s == 15)
    def _():
        pltpu.sync_copy(rows.at[pl.ds(0, 16)], acc.at[pl.ds(HALF, 16)])
    plsc.subcore_barrier()

    def chunk(i, _):
        r0 = s * TER + i * 2
        pltpu.sync_copy(src_hbm.at[pl.ds(r0, 2)], svm)
        pltpu.sync_copy(dstrel_hbm.at[c, pl.ds(r0, 2)], dvm)
        descs = [pltpu.async_copy(y_hbm.at[svm.at[j]],
                                  rows.at[pl.ds(j * 128, 128)], sem)
                 for j in range(2)]
        for d in descs:
            d.wait()
        for j in range(2):
            pltpu.sync_copy(rows.at[pl.ds(j * 128, 128)], acc.at[dvm.at[j]],
                            add=True)
        return ()
    lax.fori_loop(0, TER // 2, chunk, ())
    plsc.subcore_barrier()

    pltpu.sync_copy(acc.at[pl.ds(s * TPR, TPR)],
                    z_hbm.at[pl.ds(c * HALF + s * TPR, TPR)])


@functools.partial(
    pl.kernel,
    out_type=jax.ShapeDtypeStruct((2, GG, 64), jnp.float32),
    mesh=_mesh,
    compiler_params=pltpu.CompilerParams(use_tc_tiling_on_sc=False),
    scratch_types=[
        pltpu.VMEM((8, 128), jnp.int32),
        pltpu.VMEM((512, 64), jnp.float32),
        pltpu.VMEM_SHARED((GG + 16, 64), jnp.float32),
    ],
)
def _sc_pool(h_hbm, batch_hbm, pool_hbm, bvm, rows, pacc):
    c = lax.axis_index("c")
    s = lax.axis_index("s")
    w = c * 16 + s

    _zero_fill_rows(rows, 512)
    pltpu.sync_copy(rows.at[pl.ds(0, 32)], pacc.at[pl.ds(s * 32, 32)])

    @pl.when(s == 15)
    def _():
        pltpu.sync_copy(rows.at[pl.ds(0, 16)], pacc.at[pl.ds(GG, 16)])
    plsc.subcore_barrier()

    def block(b):
        pltpu.sync_copy(batch_hbm.at[pl.ds(b * 8, 8)], bvm)
        for k in range(2):
            pltpu.sync_copy(h_hbm.at[pl.ds(b * 1024 + k * 512, 512)], rows)
            for j in range(4):
                pltpu.sync_copy(rows.at[pl.ds(j * 128, 128)],
                                pacc.at[bvm.at[k * 4 + j]], add=True)

    block(w)

    @pl.when(w + 32 < NR // 8)
    def _():
        block(w + 32)
    plsc.subcore_barrier()

    @pl.when(s == 0)
    def _():
        pltpu.sync_copy(pacc.at[pl.ds(0, GG)], pool_hbm.at[c])


def _tc_pre_body(x_ref, deg_ref, w1_ref, y_ref):
    dinv = lax.rsqrt(deg_ref[...][:, 0:1] + 1.0)
    x = x_ref[...]
    w1 = w1_ref[...]
    xw = x[:, 0:1] * w1[0:1, :] + x[:, 1:2] * w1[1:2, :]
    y_ref[...] = dinv * xw


_tc_pre = pl.pallas_call(
    _tc_pre_body,
    out_shape=jax.ShapeDtypeStruct((NP_, 64), jnp.float32),
    grid=(98,),
    in_specs=[
        pl.BlockSpec((512, 2), lambda i: (i, 0)),
        pl.BlockSpec((512, 16), lambda i: (i, 0)),
        pl.BlockSpec((2, 64), lambda i: (0, 0)),
    ],
    out_specs=pl.BlockSpec((512, 64), lambda i: (i, 0)),
)


def _tc_mid_body(z_ref, y_ref, deg_ref, b_ref, w_ref, o_ref):
    dinv = lax.rsqrt(deg_ref[...][:, 0:1] + 1.0)
    h = jnp.maximum(dinv * (z_ref[...] + y_ref[...]) + b_ref[...], 0.0)
    o_ref[...] = dinv * jnp.dot(h, w_ref[...],
                                preferred_element_type=jnp.float32)


_tc_mid = pl.pallas_call(
    _tc_mid_body,
    out_shape=jax.ShapeDtypeStruct((NP_, 64), jnp.float32),
    grid=(98,),
    in_specs=[
        pl.BlockSpec((512, 64), lambda i: (i, 0)),
        pl.BlockSpec((512, 64), lambda i: (i, 0)),
        pl.BlockSpec((512, 16), lambda i: (i, 0)),
        pl.BlockSpec((1, 64), lambda i: (0, 0)),
        pl.BlockSpec((64, 64), lambda i: (0, 0)),
    ],
    out_specs=pl.BlockSpec((512, 64), lambda i: (i, 0)),
)


def _tc_ep_body(z_ref, y_ref, deg_ref, b_ref, o_ref):
    dinv = lax.rsqrt(deg_ref[...][:, 0:1] + 1.0)
    o_ref[...] = dinv * (z_ref[...] + y_ref[...]) + b_ref[...]


_tc_ep = pl.pallas_call(
    _tc_ep_body,
    out_shape=jax.ShapeDtypeStruct((NP_, 64), jnp.float32),
    grid=(98,),
    in_specs=[
        pl.BlockSpec((512, 64), lambda i: (i, 0)),
        pl.BlockSpec((512, 64), lambda i: (i, 0)),
        pl.BlockSpec((512, 16), lambda i: (i, 0)),
        pl.BlockSpec((1, 64), lambda i: (0, 0)),
    ],
    out_specs=pl.BlockSpec((512, 64), lambda i: (i, 0)),
)


def _tc_head_body(p_ref, cnt_ref, w1_ref, b1_ref, w2_ref, b2_ref, o_ref):
    p = p_ref[0] + p_ref[1]
    pooled = p / jnp.maximum(cnt_ref[...][:, 0:1], 1.0)
    a = jnp.maximum(
        jnp.dot(pooled, w1_ref[...], preferred_element_type=jnp.float32)
        + b1_ref[...], 0.0)
    o_ref[...] = jnp.sum(a * w2_ref[...], axis=1, keepdims=True) + b2_ref[...]


_tc_head = pl.pallas_call(
    _tc_head_body,
    out_shape=jax.ShapeDtypeStruct((GG, 1), jnp.float32),
)


def kernel(x, edge_index, batch, W1, b1, W2, b2, W3, b3, l1W, l1b, l2W, l2b):
    src = edge_index[0]
    dst = edge_index[1]
    srcp = jnp.pad(src, (0, EP - EE)).reshape(ER, 128)
    dstp = jnp.pad(dst, (0, EP - EE), constant_values=BIGDST).reshape(ER, 128)
    batp = jnp.pad(batch, (0, NP_ - NN), constant_values=GG).reshape(NR, 128)
    xp = jnp.pad(x, ((0, NP_ - NN), (0, 0)))

    dstrel, deg16, cnt16 = _sc_prep(dstp, batp)
    y1 = _tc_pre(xp, deg16, W1)
    z1 = _sc_prop(y1, srcp, dstrel)
    y2 = _tc_mid(z1, y1, deg16, b1.reshape(1, HH), W2)
    z2 = _sc_prop(y2, srcp, dstrel)
    y3 = _tc_mid(z2, y2, deg16, b2.reshape(1, HH), W3)
    z3 = _sc_prop(y3, srcp, dstrel)
    h3 = _tc_ep(z3, y3, deg16, b3.reshape(1, HH))
    pool = _sc_pool(h3, batp)
    out = _tc_head(pool, cnt16[0], l1W, l1b.reshape(1, HH // 2),
                   l2W.reshape(1, HH // 2), l2b.reshape(1, 1))
    return out.reshape(GG)


# speedup vs baseline: 10.7581x; 1.0840x over previous
import functools

import jax
import jax.numpy as jnp
from jax import lax
from jax.experimental import pallas as pl
from jax.experimental.pallas import tpu as pltpu
from jax.experimental.pallas import tpu_sc as plsc

NN = 50000
EE = 800000
HH = 64
GG = 512

HALF = 25088
NP_ = 2 * HALF
NR = NP_ // 128
TPR = 1568
EP = 802816
ER = EP // 128
TER = ER // 16
TRASH = HALF
BIGDST = 0x3FFFFF0

_mesh = plsc.VectorSubcoreMesh(core_axis_name="c", subcore_axis_name="s")


def _zero_fill_rows(ref, nrows):
    def body(i, _):
        ref[i // 4, pl.ds((i % 4) * 16, 16)] = jnp.zeros((16,), jnp.float32)
        return ()
    lax.fori_loop(0, nrows * 4, body, ())


@functools.partial(
    pl.kernel,
    out_type=(
        jax.ShapeDtypeStruct((2, ER, 128), jnp.int32),
        jax.ShapeDtypeStruct((NP_, 16), jnp.float32),
        jax.ShapeDtypeStruct((2, GG, 16), jnp.float32),
    ),
    mesh=_mesh,
    compiler_params=pltpu.CompilerParams(use_tc_tiling_on_sc=False),
    scratch_types=[
        pltpu.VMEM((8, 128), jnp.int32),
        pltpu.VMEM((8, 128), jnp.int32),
        pltpu.VMEM((32, 128), jnp.int32),
        pltpu.VMEM((128, 16), jnp.float32),
        pltpu.VMEM((176, 16), jnp.float32),
        pltpu.VMEM_SHARED((HALF + 16, 16), jnp.float32),
        pltpu.VMEM_SHARED((GG + 16, 16), jnp.float32),
    ],
)
def _sc_prep(dst_hbm, batch_hbm, dstrel_hbm, deg_hbm, cnt_hbm,
             dvm, rvm, bvm, ones, zeros, deg_acc, cnt_acc):
    c = lax.axis_index("c")
    s = lax.axis_index("s")

    def f_ones(i, _):
        ones[i, :] = jnp.full((16,), 1.0, jnp.float32)
        return ()
    lax.fori_loop(0, 128, f_ones, ())

    def f_zero(i, _):
        zeros[i, :] = jnp.zeros((16,), jnp.float32)
        return ()
    lax.fori_loop(0, 176, f_zero, ())

    off = 0
    while off < TPR:
        sz = min(176, TPR - off)
        pltpu.sync_copy(zeros.at[pl.ds(0, sz)],
                        deg_acc.at[pl.ds(s * TPR + off, sz)])
        off += sz

    pltpu.sync_copy(zeros.at[pl.ds(0, 32)], cnt_acc.at[pl.ds(s * 32, 32)])

    @pl.when(s == 15)
    def _():
        pltpu.sync_copy(zeros.at[pl.ds(0, 16)], deg_acc.at[pl.ds(HALF, 16)])
        pltpu.sync_copy(zeros.at[pl.ds(0, 16)], cnt_acc.at[pl.ds(GG, 16)])
    plsc.subcore_barrier()

    def chunk(i, _):
        r0 = s * TER + i * 8
        pltpu.sync_copy(dst_hbm.at[pl.ds(r0, 8)], dvm)
        for j in range(8):
            for k in range(8):
                d = dvm[j, pl.ds(k * 16, 16)]
                rel = d - c * HALF
                ok = (rel >= 0) & (rel < HALF)
                rvm[j, pl.ds(k * 16, 16)] = jnp.where(ok, rel, TRASH)
        pltpu.sync_copy(rvm, dstrel_hbm.at[c, pl.ds(r0, 8)])
        for j in range(8):
            pltpu.sync_copy(ones, deg_acc.at[rvm.at[j]], add=True)
        return ()
    lax.fori_loop(0, TER // 8, chunk, ())

    pltpu.sync_copy(batch_hbm.at[pl.ds(s * 24, 24)], bvm.at[pl.ds(0, 24)])
    for j in range(24):
        pltpu.sync_copy(ones, cnt_acc.at[bvm.at[j]], add=True)

    @pl.when(s == 15)
    def _():
        pltpu.sync_copy(batch_hbm.at[pl.ds(384, 8)], bvm.at[pl.ds(24, 8)])
        for j in range(24, 32):
            pltpu.sync_copy(ones, cnt_acc.at[bvm.at[j]], add=True)
    plsc.subcore_barrier()

    pltpu.sync_copy(deg_acc.at[pl.ds(s * TPR, TPR)],
                    deg_hbm.at[pl.ds(c * HALF + s * TPR, TPR)])

    @pl.when(s == 0)
    def _():
        pltpu.sync_copy(cnt_acc.at[pl.ds(0, GG)], cnt_hbm.at[c])


@functools.partial(
    pl.kernel,
    out_type=jax.ShapeDtypeStruct((NP_, 64), jnp.float32),
    mesh=_mesh,
    compiler_params=pltpu.CompilerParams(use_tc_tiling_on_sc=False),
    scratch_types=[
        pltpu.VMEM((2, 8, 128), jnp.int32),
        pltpu.VMEM((2, 8, 128), jnp.int32),
        pltpu.VMEM((2, 128, 64), jnp.float32),
        pltpu.VMEM_SHARED((HALF + 16, 64), jnp.float32),
        pltpu.SemaphoreType.DMA,
        pltpu.SemaphoreType.DMA,
    ],
)
def _sc_prop(y_hbm, src_hbm, dstrel_hbm, z_hbm, svm, dvm, rows, acc,
             gsem, isem):
    c = lax.axis_index("c")
    s = lax.axis_index("s")

    _zero_fill_rows(rows.at[0], 128)
    for k in range(12):
        pltpu.sync_copy(rows.at[0], acc.at[pl.ds(s * TPR + k * 128, 128)])
    pltpu.sync_copy(rows.at[0, pl.ds(0, 32)], acc.at[pl.ds(s * TPR + 1536, 32)])

    @pl.when(s == 15)
    def _():
        pltpu.sync_copy(rows.at[0, pl.ds(0, 16)], acc.at[pl.ds(HALF, 16)])
    plsc.subcore_barrier()

    base = s * TER

    def idx_fire(blk, slot):
        pltpu.async_copy(src_hbm.at[pl.ds(base + blk * 8, 8)],
                         svm.at[slot], isem)
        pltpu.async_copy(dstrel_hbm.at[c, pl.ds(base + blk * 8, 8)],
                         dvm.at[slot], isem)

    def idx_wait(blk, slot):
        pltpu.make_async_copy(src_hbm.at[pl.ds(base + blk * 8, 8)],
                              svm.at[slot], isem).wait()
        pltpu.make_async_copy(dstrel_hbm.at[c, pl.ds(base + blk * 8, 8)],
                              dvm.at[slot], isem).wait()

    def gather_fire(slot, j, buf):
        pltpu.async_copy(y_hbm.at[svm.at[slot, j]], rows.at[buf], gsem)

    def gather_wait(slot, j, buf):
        pltpu.make_async_copy(y_hbm.at[svm.at[slot, j]], rows.at[buf],
                              gsem).wait()

    pltpu.sync_copy(src_hbm.at[pl.ds(base, 8)], svm.at[0])
    pltpu.sync_copy(dstrel_hbm.at[c, pl.ds(base, 8)], dvm.at[0])
    idx_fire(1, 1)
    gather_fire(0, 0, 0)

    def block(B, pb):
        for j in range(8):
            cur = j % 2
            if j < 7:
                gather_fire(pb, j + 1, 1 - cur)
            else:
                @pl.when(B < 48)
                def _():
                    idx_wait(B + 1, 1 - pb)
                    gather_fire(1 - pb, 0, 1 - cur)
            gather_wait(pb, j, cur)
            pltpu.sync_copy(rows.at[cur], acc.at[dvm.at[pb, j]], add=True)

        @pl.when(B + 2 < 49)
        def _():
            idx_fire(B + 2, pb)

    def pair(i, _):
        block(2 * i, 0)
        block(2 * i + 1, 1)
        return ()
    lax.fori_loop(0, 24, pair, ())
    block(jnp.int32(48), 0)
    plsc.subcore_barrier()

    pltpu.sync_copy(acc.at[pl.ds(s * TPR, TPR)],
                    z_hbm.at[pl.ds(c * HALF + s * TPR, TPR)])


@functools.partial(
    pl.kernel,
    out_type=jax.ShapeDtypeStruct((2, GG, 64), jnp.float32),
    mesh=_mesh,
    compiler_params=pltpu.CompilerParams(use_tc_tiling_on_sc=False),
    scratch_types=[
        pltpu.VMEM((8, 128), jnp.int32),
        pltpu.VMEM((512, 64), jnp.float32),
        pltpu.VMEM_SHARED((GG + 16, 64), jnp.float32),
    ],
)
def _sc_pool(h_hbm, batch_hbm, pool_hbm, bvm, rows, pacc):
    c = lax.axis_index("c")
    s = lax.axis_index("s")
    w = c * 16 + s

    _zero_fill_rows(rows, 512)
    pltpu.sync_copy(rows.at[pl.ds(0, 32)], pacc.at[pl.ds(s * 32, 32)])

    @pl.when(s == 15)
    def _():
        pltpu.sync_copy(rows.at[pl.ds(0, 16)], pacc.at[pl.ds(GG, 16)])
    plsc.subcore_barrier()

    def block(b):
        pltpu.sync_copy(batch_hbm.at[pl.ds(b * 8, 8)], bvm)
        for k in range(2):
            pltpu.sync_copy(h_hbm.at[pl.ds(b * 1024 + k * 512, 512)], rows)
            for j in range(4):
                pltpu.sync_copy(rows.at[pl.ds(j * 128, 128)],
                                pacc.at[bvm.at[k * 4 + j]], add=True)

    block(w)

    @pl.when(w + 32 < NR // 8)
    def _():
        block(w + 32)
    plsc.subcore_barrier()

    @pl.when(s == 0)
    def _():
        pltpu.sync_copy(pacc.at[pl.ds(0, GG)], pool_hbm.at[c])


def _tc_pre_body(x_ref, deg_ref, w1_ref, y_ref):
    dinv = lax.rsqrt(deg_ref[...][:, 0:1] + 1.0)
    x = x_ref[...]
    w1 = w1_ref[...]
    xw = x[:, 0:1] * w1[0:1, :] + x[:, 1:2] * w1[1:2, :]
    y_ref[...] = dinv * xw


_tc_pre = pl.pallas_call(
    _tc_pre_body,
    out_shape=jax.ShapeDtypeStruct((NP_, 64), jnp.float32),
    grid=(98,),
    in_specs=[
        pl.BlockSpec((512, 2), lambda i: (i, 0)),
        pl.BlockSpec((512, 16), lambda i: (i, 0)),
        pl.BlockSpec((2, 64), lambda i: (0, 0)),
    ],
    out_specs=pl.BlockSpec((512, 64), lambda i: (i, 0)),
)


def _tc_mid_body(z_ref, y_ref, deg_ref, b_ref, w_ref, o_ref):
    dinv = lax.rsqrt(deg_ref[...][:, 0:1] + 1.0)
    h = jnp.maximum(dinv * (z_ref[...] + y_ref[...]) + b_ref[...], 0.0)
    o_ref[...] = dinv * jnp.dot(h, w_ref[...],
                                preferred_element_type=jnp.float32)


_tc_mid = pl.pallas_call(
    _tc_mid_body,
    out_shape=jax.ShapeDtypeStruct((NP_, 64), jnp.float32),
    grid=(98,),
    in_specs=[
        pl.BlockSpec((512, 64), lambda i: (i, 0)),
        pl.BlockSpec((512, 64), lambda i: (i, 0)),
        pl.BlockSpec((512, 16), lambda i: (i, 0)),
        pl.BlockSpec((1, 64), lambda i: (0, 0)),
        pl.BlockSpec((64, 64), lambda i: (0, 0)),
    ],
    out_specs=pl.BlockSpec((512, 64), lambda i: (i, 0)),
)


def _tc_ep_body(z_ref, y_ref, deg_ref, b_ref, o_ref):
    dinv = lax.rsqrt(deg_ref[...][:, 0:1] + 1.0)
    o_ref[...] = dinv * (z_ref[...] + y_ref[...]) + b_ref[...]


_tc_ep = pl.pallas_call(
    _tc_ep_body,
    out_shape=jax.ShapeDtypeStruct((NP_, 64), jnp.float32),
    grid=(98,),
    in_specs=[
        pl.BlockSpec((512, 64), lambda i: (i, 0)),
        pl.BlockSpec((512, 64), lambda i: (i, 0)),
        pl.BlockSpec((512, 16), lambda i: (i, 0)),
        pl.BlockSpec((1, 64), lambda i: (0, 0)),
    ],
    out_specs=pl.BlockSpec((512, 64), lambda i: (i, 0)),
)


def _tc_head_body(p_ref, cnt_ref, w1_ref, b1_ref, w2_ref, b2_ref, o_ref):
    p = p_ref[0] + p_ref[1]
    pooled = p / jnp.maximum(cnt_ref[...][:, 0:1], 1.0)
    a = jnp.maximum(
        jnp.dot(pooled, w1_ref[...], preferred_element_type=jnp.float32)
        + b1_ref[...], 0.0)
    o_ref[...] = jnp.sum(a * w2_ref[...], axis=1, keepdims=True) + b2_ref[...]


_tc_head = pl.pallas_call(
    _tc_head_body,
    out_shape=jax.ShapeDtypeStruct((GG, 1), jnp.float32),
)


def kernel(x, edge_index, batch, W1, b1, W2, b2, W3, b3, l1W, l1b, l2W, l2b):
    src = edge_index[0]
    dst = edge_index[1]
    srcp = jnp.pad(src, (0, EP - EE)).reshape(ER, 128)
    dstp = jnp.pad(dst, (0, EP - EE), constant_values=BIGDST).reshape(ER, 128)
    batp = jnp.pad(batch, (0, NP_ - NN), constant_values=GG).reshape(NR, 128)
    xp = jnp.pad(x, ((0, NP_ - NN), (0, 0)))

    dstrel, deg16, cnt16 = _sc_prep(dstp, batp)
    y1 = _tc_pre(xp, deg16, W1)
    z1 = _sc_prop(y1, srcp, dstrel)
    y2 = _tc_mid(z1, y1, deg16, b1.reshape(1, HH), W2)
    z2 = _sc_prop(y2, srcp, dstrel)
    y3 = _tc_mid(z2, y2, deg16, b2.reshape(1, HH), W3)
    z3 = _sc_prop(y3, srcp, dstrel)
    h3 = _tc_ep(z3, y3, deg16, b3.reshape(1, HH))
    pool = _sc_pool(h3, batp)
    out = _tc_head(pool, cnt16[0], l1W, l1b.reshape(1, HH // 2),
                   l2W.reshape(1, HH // 2), l2b.reshape(1, 1))
    return out.reshape(GG)
